# baseline (speedup 1.0000x reference)
import functools

import jax
import jax.numpy as jnp
from jax import lax
from jax.experimental import pallas as pl
from jax.experimental.pallas import tpu as pltpu
from jax.experimental.pallas import tpu_sc as plsc

VOCAB = 100000
EMBED_DIM = 128
HIDDEN_DIM = 256
NUM_LABELS = 1000
BAG_LEN = 16384

NUM_CORES = 2
NUM_SUBCORES = 16
NUM_WORKERS = NUM_CORES * NUM_SUBCORES
BPW = BAG_LEN // NUM_WORKERS
CHUNK = 64
NCHUNK = BPW // CHUNK
NBUF = 8
NLANE = EMBED_DIM // 16


UNROLL = 4


def _sc_gather_sum_body(idx_hbm, table_hbm, out_hbm,
                        idx_v, rows0, rows1, rows2, rows3,
                        rows4, rows5, rows6, rows7, acc_v,
                        sem0, sem1, sem2, sem3, sem4, sem5, sem6, sem7):
    wid = lax.axis_index("s") * NUM_CORES + lax.axis_index("c")
    base = wid * BPW

    rows = (rows0, rows1, rows2, rows3, rows4, rows5, rows6, rows7)
    sems = (sem0, sem1, sem2, sem3, sem4, sem5, sem6, sem7)

    def fire(c):
        return pltpu.async_copy(
            table_hbm.at[idx_v.at[pl.ds(c * CHUNK, CHUNK)]],
            rows[c % NBUF], sems[c % NBUF])

    pltpu.sync_copy(idx_hbm.at[pl.ds(base, CHUNK)], idx_v.at[pl.ds(0, CHUNK)])
    cps = {0: fire(0)}
    pltpu.sync_copy(idx_hbm.at[pl.ds(base + CHUNK, BPW - CHUNK)],
                    idx_v.at[pl.ds(CHUNK, BPW - CHUNK)])
    for c in range(1, NBUF - 1):
        cps[c] = fire(c)

    acc = tuple(jnp.zeros((16,), jnp.float32) for _ in range(NLANE))
    for c in range(NCHUNK):
        nxt = c + NBUF - 1
        if nxt < NCHUNK:
            cps[nxt] = fire(nxt)
        cps[c].wait()
        r = rows[c % NBUF]

        def row_body(i, a, r=r):
            return tuple(a[j] + r[i, pl.ds(j * 16, 16)]
                         for j in range(NLANE))

        acc = lax.fori_loop(0, CHUNK, row_body, acc)

    for j in range(NLANE):
        acc_v[0, pl.ds(j * 16, 16)] = acc[j]
    pltpu.sync_copy(acc_v, out_hbm.at[pl.ds(wid, 1)])


_sc_gather_sum = functools.partial(
    pl.kernel,
    out_type=jax.ShapeDtypeStruct((NUM_WORKERS, EMBED_DIM), jnp.float32),
    mesh=plsc.VectorSubcoreMesh(core_axis_name="c", subcore_axis_name="s"),
    scratch_types=[
        pltpu.VMEM((BPW,), jnp.int32),
        pltpu.VMEM((CHUNK, EMBED_DIM), jnp.float32),
        pltpu.VMEM((CHUNK, EMBED_DIM), jnp.float32),
        pltpu.VMEM((CHUNK, EMBED_DIM), jnp.float32),
        pltpu.VMEM((CHUNK, EMBED_DIM), jnp.float32),
        pltpu.VMEM((CHUNK, EMBED_DIM), jnp.float32),
        pltpu.VMEM((CHUNK, EMBED_DIM), jnp.float32),
        pltpu.VMEM((CHUNK, EMBED_DIM), jnp.float32),
        pltpu.VMEM((CHUNK, EMBED_DIM), jnp.float32),
        pltpu.VMEM((1, EMBED_DIM), jnp.float32),
        pltpu.SemaphoreType.DMA,
        pltpu.SemaphoreType.DMA,
        pltpu.SemaphoreType.DMA,
        pltpu.SemaphoreType.DMA,
        pltpu.SemaphoreType.DMA,
        pltpu.SemaphoreType.DMA,
        pltpu.SemaphoreType.DMA,
        pltpu.SemaphoreType.DMA,

    ],
)(_sc_gather_sum_body)


def _mlp_body(p_ref, wh_ref, bh_ref, wo_ref, bo_ref, o_ref):
    emb = jnp.sum(p_ref[...], axis=0, keepdims=True)
    h = jnp.dot(emb, wh_ref[...], preferred_element_type=jnp.float32)
    h = jnp.maximum(h + bh_ref[...], 0.0)
    o = jnp.dot(h, wo_ref[...], preferred_element_type=jnp.float32)
    o_ref[...] = o + bo_ref[...]


_mlp = pl.pallas_call(
    _mlp_body,
    out_shape=jax.ShapeDtypeStruct((1, NUM_LABELS), jnp.float32),
)


@jax.jit
def kernel(X, table, W_hidden, b_hidden, W_out, b_out):
    idx = X.astype(jnp.int32)
    partials = _sc_gather_sum(idx, table)
    return _mlp(partials, W_hidden, b_hidden.reshape(1, HIDDEN_DIM),
                W_out, b_out.reshape(1, NUM_LABELS))


# speedup vs baseline: 1.0297x; 1.0297x over previous
import functools

import jax
import jax.numpy as jnp
from jax import lax
from jax.experimental import pallas as pl
from jax.experimental.pallas import tpu as pltpu
from jax.experimental.pallas import tpu_sc as plsc

VOCAB = 100000
EMBED_DIM = 128
HIDDEN_DIM = 256
NUM_LABELS = 1000
BAG_LEN = 16384

NUM_CORES = 2
NUM_SUBCORES = 16
NUM_WORKERS = NUM_CORES * NUM_SUBCORES
BPW = BAG_LEN // NUM_WORKERS
CHUNK = 128
NCHUNK = BPW // CHUNK
NBUF = 2
NLANE = EMBED_DIM // 16


UNROLL = 4


def _sc_gather_sum_body(idx_hbm, table_hbm, out_hbm,
                        idx_v, rows0, rows1, acc_v,
                        sem0, sem1):
    wid = lax.axis_index("s") * NUM_CORES + lax.axis_index("c")
    base = wid * BPW

    rows = (rows0, rows1)
    sems = (sem0, sem1)

    def fire(c):
        return pltpu.async_copy(
            table_hbm.at[idx_v.at[pl.ds(c * CHUNK, CHUNK)]],
            rows[c % NBUF], sems[c % NBUF])

    pltpu.sync_copy(idx_hbm.at[pl.ds(base, CHUNK)], idx_v.at[pl.ds(0, CHUNK)])
    cps = {0: fire(0)}
    pltpu.sync_copy(idx_hbm.at[pl.ds(base + CHUNK, BPW - CHUNK)],
                    idx_v.at[pl.ds(CHUNK, BPW - CHUNK)])
    for c in range(1, NBUF - 1):
        cps[c] = fire(c)

    acc = tuple(jnp.zeros((16,), jnp.float32) for _ in range(NLANE))
    for c in range(NCHUNK):
        nxt = c + NBUF - 1
        if nxt < NCHUNK:
            cps[nxt] = fire(nxt)
        cps[c].wait()
        r = rows[c % NBUF]

        def row_body(i, a, r=r):
            return tuple(a[j] + r[i, pl.ds(j * 16, 16)]
                         for j in range(NLANE))

        acc = lax.fori_loop(0, CHUNK, row_body, acc, unroll=UNROLL)

    for j in range(NLANE):
        acc_v[0, pl.ds(j * 16, 16)] = acc[j]
    pltpu.sync_copy(acc_v, out_hbm.at[pl.ds(wid, 1)])


_sc_gather_sum = functools.partial(
    pl.kernel,
    out_type=jax.ShapeDtypeStruct((NUM_WORKERS, EMBED_DIM), jnp.float32),
    mesh=plsc.VectorSubcoreMesh(core_axis_name="c", subcore_axis_name="s"),
    scratch_types=[
        pltpu.VMEM((BPW,), jnp.int32),
        pltpu.VMEM((CHUNK, EMBED_DIM), jnp.float32),
        pltpu.VMEM((CHUNK, EMBED_DIM), jnp.float32),
        pltpu.VMEM((1, EMBED_DIM), jnp.float32),
        pltpu.SemaphoreType.DMA,
        pltpu.SemaphoreType.DMA,
    ],
)(_sc_gather_sum_body)


def _mlp_body(p_ref, wh_ref, bh_ref, wot_ref, bo_ref, o_ref):
    emb = jnp.sum(p_ref[...], axis=0, keepdims=True)
    h = jnp.dot(emb, wh_ref[...], preferred_element_type=jnp.float32)
    h = jnp.maximum(h + bh_ref[...], 0.0)
    o = lax.dot_general(h, wot_ref[...],
                        dimension_numbers=(((1,), (1,)), ((), ())),
                        preferred_element_type=jnp.float32)
    o_ref[...] = o + bo_ref[...]


_mlp = pl.pallas_call(
    _mlp_body,
    out_shape=jax.ShapeDtypeStruct((1, NUM_LABELS), jnp.float32),
)


@jax.jit
def kernel(X, table, W_hidden, b_hidden, W_out, b_out):
    idx = X.astype(jnp.int32)
    partials = _sc_gather_sum(idx, table)
    return _mlp(partials, W_hidden, b_hidden.reshape(1, HIDDEN_DIM),
                W_out.T, b_out.reshape(1, NUM_LABELS))


# speedup vs baseline: 1.0481x; 1.0179x over previous
import functools

import jax
import jax.numpy as jnp
from jax import lax
from jax.experimental import pallas as pl
from jax.experimental.pallas import tpu as pltpu
from jax.experimental.pallas import tpu_sc as plsc

VOCAB = 100000
EMBED_DIM = 128
HIDDEN_DIM = 256
NUM_LABELS = 1000
BAG_LEN = 16384

NUM_CORES = 2
NUM_SUBCORES = 16
NUM_WORKERS = NUM_CORES * NUM_SUBCORES
BPW = BAG_LEN // NUM_WORKERS
CHUNK = 128
NCHUNK = BPW // CHUNK
NBUF = 3
NLANE = EMBED_DIM // 16


UNROLL = 4


def _sc_gather_sum_body(idx_hbm, table_hbm, out_hbm,
                        idx_v, rows0, rows1, rows2, acc_v,
                        sem0, sem1, sem2):
    wid = lax.axis_index("s") * NUM_CORES + lax.axis_index("c")
    base = wid * BPW

    rows = (rows0, rows1, rows2)
    sems = (sem0, sem1, sem2)

    def fire(c):
        return pltpu.async_copy(
            table_hbm.at[idx_v.at[pl.ds(c * CHUNK, CHUNK)]],
            rows[c % NBUF], sems[c % NBUF])

    pltpu.sync_copy(idx_hbm.at[pl.ds(base, CHUNK)], idx_v.at[pl.ds(0, CHUNK)])
    cps = {0: fire(0)}
    pltpu.sync_copy(idx_hbm.at[pl.ds(base + CHUNK, BPW - CHUNK)],
                    idx_v.at[pl.ds(CHUNK, BPW - CHUNK)])
    for c in range(1, NBUF - 1):
        cps[c] = fire(c)

    acc = tuple(jnp.zeros((16,), jnp.float32) for _ in range(NLANE))
    for c in range(NCHUNK):
        nxt = c + NBUF - 1
        if nxt < NCHUNK:
            cps[nxt] = fire(nxt)
        cps[c].wait()
        r = rows[c % NBUF]

        def row_body(i, a, r=r):
            return tuple(a[j] + r[i, pl.ds(j * 16, 16)]
                         for j in range(NLANE))

        acc = lax.fori_loop(0, CHUNK, row_body, acc, unroll=UNROLL)

    for j in range(NLANE):
        acc_v[0, pl.ds(j * 16, 16)] = acc[j]
    pltpu.sync_copy(acc_v, out_hbm.at[pl.ds(wid, 1)])


_sc_gather_sum = functools.partial(
    pl.kernel,
    out_type=jax.ShapeDtypeStruct((NUM_WORKERS, EMBED_DIM), jnp.float32),
    mesh=plsc.VectorSubcoreMesh(core_axis_name="c", subcore_axis_name="s"),
    scratch_types=[
        pltpu.VMEM((BPW,), jnp.int32),
        pltpu.VMEM((CHUNK, EMBED_DIM), jnp.float32),
        pltpu.VMEM((CHUNK, EMBED_DIM), jnp.float32),
        pltpu.VMEM((CHUNK, EMBED_DIM), jnp.float32),
        pltpu.VMEM((1, EMBED_DIM), jnp.float32),
        pltpu.SemaphoreType.DMA,
        pltpu.SemaphoreType.DMA,
        pltpu.SemaphoreType.DMA,
    ],
)(_sc_gather_sum_body)


def _mlp_body(p_ref, wh_ref, bh_ref, wot_ref, bo_ref, o_ref):
    emb = jnp.sum(p_ref[...], axis=0, keepdims=True)
    h = jnp.dot(emb, wh_ref[...], preferred_element_type=jnp.float32)
    h = jnp.maximum(h + bh_ref[...], 0.0)
    o = lax.dot_general(h, wot_ref[...],
                        dimension_numbers=(((1,), (1,)), ((), ())),
                        preferred_element_type=jnp.float32)
    o_ref[...] = o + bo_ref[...]


_mlp = pl.pallas_call(
    _mlp_body,
    out_shape=jax.ShapeDtypeStruct((1, NUM_LABELS), jnp.float32),
)


@jax.jit
def kernel(X, table, W_hidden, b_hidden, W_out, b_out):
    idx = X.astype(jnp.int32)
    partials = _sc_gather_sum(idx, table)
    return _mlp(partials, W_hidden, b_hidden.reshape(1, HIDDEN_DIM),
                W_out.T, b_out.reshape(1, NUM_LABELS))
